# baseline (speedup 1.0000x reference)
import functools

import jax
import jax.numpy as jnp
from jax import lax
from jax.experimental import pallas as pl
from jax.experimental.pallas import tpu as pltpu
from jax.experimental.pallas import tpu_sc as plsc

D = 64
VOCAB_HALF = 500000
SCALE = 8.0
B_TOTAL = 4096 * 200
NC, NS, L = 2, 16, 16
NW = NC * NS
B_PER_W = B_TOTAL // NW

NBUF = 4
CHUNK = 256
IDX_PER_STREAM = 128
NSTREAM = CHUNK // IDX_PER_STREAM
N_CHUNKS = B_PER_W // CHUNK

_mesh = plsc.VectorSubcoreMesh(
    core_axis_name="c", subcore_axis_name="s", num_cores=NC
)


@functools.partial(
    pl.kernel,
    mesh=_mesh,
    out_type=jax.ShapeDtypeStruct((B_TOTAL, D), jnp.float32),
    scratch_types=[
        pltpu.VMEM((B_PER_W,), jnp.int32),
        pltpu.VMEM((NBUF, CHUNK, D), jnp.float32),
        pltpu.SemaphoreType.DMA((NBUF,)),
        pltpu.SemaphoreType.DMA((NBUF,)),
    ],
    compiler_params=pltpu.CompilerParams(use_tc_tiling_on_sc=False),
)
def _emb_lookup(x_hbm, table_hbm, out_hbm, idx_v, rows_v, gsem, osem):
    wid = lax.axis_index("s") * NC + lax.axis_index("c")
    base = wid * B_PER_W
    pltpu.sync_copy(x_hbm.at[pl.ds(base, B_PER_W)], idx_v)

    def remap_body(i, _):
        sl = pl.ds(i * L, L)
        v = idx_v[sl]
        two = v + v
        idx_v[sl] = jnp.where(v < VOCAB_HALF, two, two - (2 * VOCAB_HALF - 1))
        return 0

    lax.fori_loop(0, B_PER_W // L, remap_body, 0)

    def fire_gather(c, b):
        for s in range(NSTREAM):
            idx_sl = idx_v.at[pl.ds(c * CHUNK + s * IDX_PER_STREAM, IDX_PER_STREAM)]
            dst = rows_v.at[b].at[pl.ds(s * IDX_PER_STREAM, IDX_PER_STREAM)]
            pltpu.async_copy(table_hbm.at[idx_sl], dst, gsem.at[b])

    def wait_gather(b):
        pltpu.make_async_copy(
            out_hbm.at[pl.ds(0, CHUNK)], rows_v.at[b], gsem.at[b]
        ).wait()

    def fire_scatter(c, b):
        pltpu.async_copy(
            rows_v.at[b], out_hbm.at[pl.ds(base + c * CHUNK, CHUNK)], osem.at[b]
        )

    def wait_scatter(b):
        pltpu.make_async_copy(
            out_hbm.at[pl.ds(0, CHUNK)], rows_v.at[b], osem.at[b]
        ).wait()

    def scale_buf(b):
        def row_body(i, _):
            for r in range(2):
                for j in range(D // L):
                    sl = pl.ds(j * L, L)
                    rows_v[b, i * 2 + r, sl] = rows_v[b, i * 2 + r, sl] * SCALE
            return 0

        lax.fori_loop(0, CHUNK // 2, row_body, 0)

    for c in range(NBUF - 1):
        fire_gather(c, c)

    def outer(g, _):
        for b in range(NBUF):
            c = g * NBUF + b
            cf = c + NBUF - 1
            bf = (b + NBUF - 1) % NBUF

            @pl.when(cf < N_CHUNKS)
            def _():
                @pl.when(c >= 1)
                def _():
                    wait_scatter(bf)

                fire_gather(cf, bf)

            wait_gather(b)
            fire_scatter(c, b)
        return 0

    lax.fori_loop(0, N_CHUNKS // NBUF, outer, 0)
    for b in range(NBUF):
        wait_scatter(b)


ROWS_BLK = 1000


def _depad_body(a_ref, b_ref, o_ref):
    o_ref[:, 0:D] = a_ref[...] * SCALE
    o_ref[:, D : 2 * D] = b_ref[...] * SCALE


_depad = pl.pallas_call(
    _depad_body,
    grid=(VOCAB_HALF // ROWS_BLK,),
    in_specs=[
        pl.BlockSpec((ROWS_BLK, D), lambda i: (i, 0)),
        pl.BlockSpec((ROWS_BLK, D), lambda i: (i + VOCAB_HALF // ROWS_BLK, 0)),
    ],
    out_specs=pl.BlockSpec((ROWS_BLK, 2 * D), lambda i: (i, 0)),
    out_shape=jax.ShapeDtypeStruct((VOCAB_HALF, 2 * D), jnp.float32),
)


def kernel(x, table):
    t_lin = jnp.reshape(_depad(table, table), (2 * VOCAB_HALF, D))
    out = _emb_lookup(x.reshape(B_TOTAL), t_lin)
    return jnp.reshape(out, (4096, 200, D))


# speedup vs baseline: 1.1577x; 1.1577x over previous
import functools

import jax
import jax.numpy as jnp
from jax import lax
from jax.experimental import pallas as pl
from jax.experimental.pallas import tpu as pltpu
from jax.experimental.pallas import tpu_sc as plsc

D = 64
VOCAB_HALF = 500000
SCALE = 8.0
B_TOTAL = 4096 * 200
NC, NS, L = 2, 16, 16
NW = NC * NS
B_PER_W = B_TOTAL // NW

NBUF = 4
CHUNK = 256
IDX_PER_STREAM = 128
NSTREAM = CHUNK // IDX_PER_STREAM
N_CHUNKS = B_PER_W // CHUNK

_mesh = plsc.VectorSubcoreMesh(
    core_axis_name="c", subcore_axis_name="s", num_cores=NC
)


@functools.partial(
    pl.kernel,
    mesh=_mesh,
    out_type=jax.ShapeDtypeStruct((B_TOTAL, D), jnp.float32),
    scratch_types=[
        pltpu.VMEM((B_PER_W,), jnp.int32),
        pltpu.VMEM((NBUF, CHUNK, D), jnp.float32),
        pltpu.SemaphoreType.DMA((NBUF,)),
        pltpu.SemaphoreType.DMA((NBUF,)),
    ],
    compiler_params=pltpu.CompilerParams(use_tc_tiling_on_sc=False),
)
def _emb_lookup(x_hbm, table_hbm, out_hbm, idx_v, rows_v, gsem, osem):
    wid = lax.axis_index("s") * NC + lax.axis_index("c")
    base = wid * B_PER_W
    pltpu.sync_copy(x_hbm.at[pl.ds(base, B_PER_W)], idx_v)

    def remap_body(i, _):
        sl = pl.ds(i * L, L)
        v = idx_v[sl]
        two = v + v
        idx_v[sl] = jnp.where(v < VOCAB_HALF, two, two - (2 * VOCAB_HALF - 1))
        return 0

    lax.fori_loop(0, B_PER_W // L, remap_body, 0)

    def fire_gather(c, b):
        for s in range(NSTREAM):
            idx_sl = idx_v.at[pl.ds(c * CHUNK + s * IDX_PER_STREAM, IDX_PER_STREAM)]
            dst = rows_v.at[b].at[pl.ds(s * IDX_PER_STREAM, IDX_PER_STREAM)]
            pltpu.async_copy(table_hbm.at[idx_sl], dst, gsem.at[b])

    def wait_gather(b):
        pltpu.make_async_copy(
            out_hbm.at[pl.ds(0, CHUNK)], rows_v.at[b], gsem.at[b]
        ).wait()

    def fire_scatter(c, b):
        pltpu.async_copy(
            rows_v.at[b], out_hbm.at[pl.ds(base + c * CHUNK, CHUNK)], osem.at[b]
        )

    def wait_scatter(b):
        pltpu.make_async_copy(
            out_hbm.at[pl.ds(0, CHUNK)], rows_v.at[b], osem.at[b]
        ).wait()

    def scale_buf(b):
        def row_body(i, _):
            for r in range(2):
                for j in range(D // L):
                    sl = pl.ds(j * L, L)
                    rows_v[b, i * 2 + r, sl] = rows_v[b, i * 2 + r, sl] * SCALE
            return 0

        lax.fori_loop(0, CHUNK // 2, row_body, 0)

    for c in range(NBUF - 1):
        fire_gather(c, c)

    def outer(g, _):
        for b in range(NBUF):
            c = g * NBUF + b
            cf = c + NBUF - 1
            bf = (b + NBUF - 1) % NBUF

            @pl.when(cf < N_CHUNKS)
            def _():
                @pl.when(c >= 1)
                def _():
                    wait_scatter(bf)

                fire_gather(cf, bf)

            wait_gather(b)
            fire_scatter(c, b)
        return 0

    lax.fori_loop(0, N_CHUNKS // NBUF, outer, 0)
    for b in range(NBUF):
        wait_scatter(b)


ROWS_BLK = 5000


def _depad_body(a_ref, b_ref, o_ref):
    o_ref[:, 0:D] = a_ref[...] * SCALE
    o_ref[:, D : 2 * D] = b_ref[...] * SCALE


_depad = pl.pallas_call(
    _depad_body,
    grid=(VOCAB_HALF // ROWS_BLK,),
    in_specs=[
        pl.BlockSpec((ROWS_BLK, D), lambda i: (i, 0)),
        pl.BlockSpec((ROWS_BLK, D), lambda i: (i + VOCAB_HALF // ROWS_BLK, 0)),
    ],
    out_specs=pl.BlockSpec((ROWS_BLK, 2 * D), lambda i: (i, 0)),
    out_shape=jax.ShapeDtypeStruct((VOCAB_HALF, 2 * D), jnp.float32),
)


def kernel(x, table):
    t_lin = jnp.reshape(_depad(table, table), (2 * VOCAB_HALF, D))
    out = _emb_lookup(x.reshape(B_TOTAL), t_lin)
    return jnp.reshape(out, (4096, 200, D))
